# baseline (speedup 1.0000x reference)
import functools

import jax
import jax.numpy as jnp
from jax import lax
from jax.experimental import pallas as pl
from jax.experimental.pallas import tpu as pltpu
from jax.experimental.pallas import tpu_sc as plsc

N = 5850
E = 93600
NPAD = 5888
RNG = 368
EPT = 5856
EPT_LAST = 5760
F32 = jnp.float32

R_DEG = 0
R_S1 = 1
R_S2 = 4
R_GP = 9
R_H = 12
NROWS = 17

_mesh = plsc.VectorSubcoreMesh(core_axis_name="c", subcore_axis_name="s",
                               num_cores=1)


@functools.partial(
    pl.kernel,
    out_type=jax.ShapeDtypeStruct((58880,), F32),
    mesh=_mesh,
    compiler_params=pltpu.CompilerParams(needs_layout_passes=False),
    scratch_types=[
        pltpu.VMEM((1104,), F32),
        pltpu.VMEM((368,), F32),
        pltpu.VMEM((1104,), F32),
        pltpu.VMEM((1104,), F32),
        pltpu.VMEM((1840,), F32),
        pltpu.VMEM((1840,), F32),
        pltpu.VMEM((5856,), jnp.int32),
        pltpu.VMEM((5856,), jnp.int32),
        pltpu.VMEM((29440,), F32),
        pltpu.VMEM((17664,), F32),
        pltpu.VMEM((29440,), F32),
        pltpu.VMEM((352,), F32),
        pltpu.VMEM((3680,), F32),
        pltpu.SemaphoreType.DMA,
        pltpu.VMEM_SHARED((NROWS * NPAD,), F32),
    ],
)
def _sc_graph(pos_hbm, esrc_hbm, edst_hbm, smalls_hbm, y_hbm,
              posr_v, dinv_v, gp_v, s1r_v, h_v, s2r_v, src_v, dst_v, msg_v,
              gpf_v, hf_v, smalls_v, y_v, sem, shared_s):
    t = lax.axis_index("s")
    zero16 = jnp.zeros((16,), F32)
    base_n = t * RNG

    def sget(idx):
        return smalls_v[pl.ds((idx // 16) * 16, 16)][idx % 16]

    def srow(r, off, size):
        return shared_s.at[pl.ds(r * NPAD + off, size)]

    pltpu.sync_copy(smalls_hbm, smalls_v)

    @pl.when(t < 15)
    def _():
        pltpu.sync_copy(pos_hbm.at[pl.ds(t * 1104, 1104)], posr_v)
        pltpu.sync_copy(esrc_hbm.at[pl.ds(t * EPT, EPT)], src_v)
        pltpu.sync_copy(edst_hbm.at[pl.ds(t * EPT, EPT)], dst_v)

    @pl.when(t == 15)
    def _():
        pltpu.sync_copy(pos_hbm.at[pl.ds(15 * 1104, 990)],
                        posr_v.at[pl.ds(0, 990)])
        pltpu.sync_copy(esrc_hbm.at[pl.ds(15 * EPT, EPT_LAST)],
                        src_v.at[pl.ds(0, EPT_LAST)])
        pltpu.sync_copy(edst_hbm.at[pl.ds(15 * EPT, EPT_LAST)],
                        dst_v.at[pl.ds(0, EPT_LAST)])

    def _zero_body(i, carry):
        msg_v[pl.ds(i * 16, 16)] = zero16
        return carry
    lax.fori_loop(0, 9 * NPAD // (16 * 16), _zero_body, None)

    zchunk = 9 * NPAD // 16
    pltpu.sync_copy(msg_v.at[pl.ds(0, zchunk)],
                    shared_s.at[pl.ds(t * zchunk, zchunk)])

    plsc.subcore_barrier()

    def _ones_body(i, carry):
        msg_v[pl.ds(i * 16, 16)] = zero16 + 1.0
        return carry
    lax.fori_loop(0, EPT // 16, _ones_body, None)

    @pl.when(t < 15)
    def _():
        pltpu.sync_copy(msg_v.at[pl.ds(0, EPT)], srow(R_DEG, 0, NPAD).at[dst_v],
                        add=True)

    @pl.when(t == 15)
    def _():
        pltpu.sync_copy(msg_v.at[pl.ds(0, EPT_LAST)],
                        srow(R_DEG, 0, NPAD).at[dst_v.at[pl.ds(0, EPT_LAST)]],
                        add=True)

    plsc.subcore_barrier()

    pltpu.sync_copy(srow(R_DEG, base_n, RNG), dinv_v)

    lane = lax.iota(jnp.int32, 16)

    def _dinv_body(i, carry):
        d = dinv_v[pl.ds(i * 16, 16)] + 1.0
        bits = lax.bitcast_convert_type(d, jnp.int32)
        y = lax.bitcast_convert_type(
            jnp.int32(0x5F3759DF) - lax.shift_right_logical(bits, 1), F32)
        for _ in range(3):
            y = y * (1.5 - 0.5 * d * y * y)
        dinv_v[pl.ds(i * 16, 16)] = y
        nloc3 = (i * 16 + lane) * 3
        for c in range(3):
            pc = plsc.load_gather(posr_v, [nloc3 + c])
            gp_v[pl.ds(c * RNG + i * 16, 16)] = y * pc
        return carry
    lax.fori_loop(0, RNG // 16, _dinv_body, None)

    for c in range(3):
        pltpu.sync_copy(gp_v.at[pl.ds(c * RNG, RNG)],
                        srow(R_GP + c, base_n, RNG))

    plsc.subcore_barrier()

    nvec = jnp.where(t == 15, EPT_LAST // 16, EPT // 16)

    def _msg_pass(tbl_v, rows_to, nfeat):
        def _g_body(i, carry):
            s16 = src_v[pl.ds(i * 16, 16)]
            for c in range(nfeat):
                v = plsc.load_gather(tbl_v, [s16 + c * NPAD])
                msg_v[pl.ds(c * NPAD + i * 16, 16)] = v
            return carry
        lax.fori_loop(0, nvec, _g_body, None)

        def sdst(c, last):
            idx = dst_v.at[pl.ds(0, EPT_LAST)] if last else dst_v
            return srow(rows_to + c, 0, NPAD).at[idx]

        def mrow(c, last):
            return msg_v.at[pl.ds(c * NPAD, EPT_LAST if last else EPT)]

        @pl.when(t < 15)
        def _():
            for c in range(nfeat):
                pltpu.async_copy(mrow(c, False), sdst(c, False), sem, add=True)
            for c in range(nfeat):
                pltpu.make_async_copy(mrow(c, False), sdst(c, False),
                                      sem).wait()

        @pl.when(t == 15)
        def _():
            for c in range(nfeat):
                pltpu.async_copy(mrow(c, True), sdst(c, True), sem, add=True)
            for c in range(nfeat):
                pltpu.make_async_copy(mrow(c, True), sdst(c, True), sem).wait()

    for c in range(3):
        pltpu.sync_copy(srow(R_GP + c, 0, NPAD), gpf_v.at[pl.ds(c * NPAD, NPAD)])
    _msg_pass(gpf_v, R_S1, 3)
    plsc.subcore_barrier()

    for c in range(3):
        pltpu.sync_copy(srow(R_S1 + c, base_n, RNG),
                        s1r_v.at[pl.ds(c * RNG, RNG)])

    def _x1_body(i, carry):
        dv = dinv_v[pl.ds(i * 16, 16)]
        ts = []
        for c in range(3):
            ts.append(dv * (s1r_v[pl.ds(c * RNG + i * 16, 16)]
                            + gp_v[pl.ds(c * RNG + i * 16, 16)]))
        for j in range(5):
            acc = sget(15 + j) + ts[0] * sget(j) \
                + ts[1] * sget(5 + j) + ts[2] * sget(10 + j)
            h_v[pl.ds(j * RNG + i * 16, 16)] = dv * jnp.maximum(acc, 0.01 * acc)
        return carry
    lax.fori_loop(0, RNG // 16, _x1_body, None)

    for c in range(5):
        pltpu.sync_copy(h_v.at[pl.ds(c * RNG, RNG)],
                        srow(R_H + c, base_n, RNG))

    plsc.subcore_barrier()

    for c in range(5):
        pltpu.sync_copy(srow(R_H + c, 0, NPAD), hf_v.at[pl.ds(c * NPAD, NPAD)])
    _msg_pass(hf_v, R_S2, 5)
    plsc.subcore_barrier()

    for c in range(5):
        pltpu.sync_copy(srow(R_S2 + c, base_n, RNG),
                        s2r_v.at[pl.ds(c * RNG, RNG)])

    def _fin_body(i, carry):
        dv = dinv_v[pl.ds(i * 16, 16)]
        ts = []
        for c in range(5):
            ts.append(dv * (s2r_v[pl.ds(c * RNG + i * 16, 16)]
                            + h_v[pl.ds(c * RNG + i * 16, 16)]))
        x2 = []
        for j in range(20):
            acc = sget(120 + j)
            for c in range(5):
                acc = acc + ts[c] * sget(20 + c * 20 + j)
            x2.append(jnp.maximum(acc, 0.01 * acc))
        nl10 = (i * 16 + lane) * 10
        for k in range(10):
            acc = sget(340 + k)
            for j in range(20):
                acc = acc + x2[j] * sget(140 + j * 10 + k)
            yv = jnp.maximum(acc, 0.01 * acc)
            plsc.store_scatter(y_v, [nl10 + k], yv)
        return carry
    lax.fori_loop(0, RNG // 16, _fin_body, None)

    pltpu.sync_copy(y_v, y_hbm.at[pl.ds(t * 3680, 3680)])


_RB = 4500


def _fc2_body(x_ref, w_ref, b_ref, o_ref):
    acc = b_ref[...]
    for i in range(13):
        acc = acc + jnp.dot(x_ref[i:i + 1, :], w_ref[_RB * i:_RB * (i + 1), :],
                            preferred_element_type=jnp.float32)
    o_ref[...] = jnp.maximum(acc, 0.01 * acc)


def _fc2(xf, W, b):
    out = pl.pallas_call(
        _fc2_body,
        out_shape=jax.ShapeDtypeStruct((1, 100), jnp.float32),
    )(xf, W, b.reshape(1, 100))
    return out[0]


def kernel(pos, edge_index, W1, b1, W2, b2, fc1_W, fc1_b, fc2_W, fc2_b):
    smalls = jnp.concatenate([
        W1.reshape(-1), b1, W2.reshape(-1), b2,
        fc1_W.reshape(-1), fc1_b, jnp.zeros((2,), jnp.float32)])
    y = _sc_graph(pos.reshape(-1), edge_index[0], edge_index[1], smalls)
    xf = y[:58500].reshape(13, 4500)
    return _fc2(xf, fc2_W, fc2_b)


# speedup vs baseline: 1.0367x; 1.0367x over previous
import functools

import jax
import jax.numpy as jnp
from jax import lax
from jax.experimental import pallas as pl
from jax.experimental.pallas import tpu as pltpu
from jax.experimental.pallas import tpu_sc as plsc

N = 5850
E = 93600
NPAD = 5888
RNG = 368
EPT = 5856
EPT_LAST = 5760
F32 = jnp.float32

R_DEG = 0
R_S1 = 1
R_S2 = 4
R_GP = 9
R_H = 12
NROWS = 17

_mesh = plsc.VectorSubcoreMesh(core_axis_name="c", subcore_axis_name="s",
                               num_cores=1)


@functools.partial(
    pl.kernel,
    out_type=jax.ShapeDtypeStruct((58880,), F32),
    mesh=_mesh,
    compiler_params=pltpu.CompilerParams(needs_layout_passes=False),
    scratch_types=[
        pltpu.VMEM((1104,), F32),
        pltpu.VMEM((368,), F32),
        pltpu.VMEM((1104,), F32),
        pltpu.VMEM((1104,), F32),
        pltpu.VMEM((1840,), F32),
        pltpu.VMEM((1840,), F32),
        pltpu.VMEM((5856,), jnp.int32),
        pltpu.VMEM((5856,), jnp.int32),
        pltpu.VMEM((29440,), F32),
        pltpu.VMEM((352,), F32),
        pltpu.VMEM((3680,), F32),
        pltpu.SemaphoreType.DMA,
        pltpu.VMEM_SHARED((NROWS * NPAD,), F32),
    ],
)
def _sc_graph(pos_hbm, esrc_hbm, edst_hbm, smalls_hbm, y_hbm,
              posr_v, dinv_v, gp_v, s1r_v, h_v, s2r_v, src_v, dst_v, msg_v,
              smalls_v, y_v, sem, shared_s):
    t = lax.axis_index("s")
    zero16 = jnp.zeros((16,), F32)
    base_n = t * RNG

    def sget(idx):
        return smalls_v[pl.ds((idx // 16) * 16, 16)][idx % 16]

    def srow(r, off, size):
        return shared_s.at[pl.ds(r * NPAD + off, size)]

    pltpu.sync_copy(smalls_hbm, smalls_v)

    @pl.when(t < 15)
    def _():
        pltpu.sync_copy(pos_hbm.at[pl.ds(t * 1104, 1104)], posr_v)
        pltpu.sync_copy(esrc_hbm.at[pl.ds(t * EPT, EPT)], src_v)
        pltpu.sync_copy(edst_hbm.at[pl.ds(t * EPT, EPT)], dst_v)

    @pl.when(t == 15)
    def _():
        pltpu.sync_copy(pos_hbm.at[pl.ds(15 * 1104, 990)],
                        posr_v.at[pl.ds(0, 990)])
        pltpu.sync_copy(esrc_hbm.at[pl.ds(15 * EPT, EPT_LAST)],
                        src_v.at[pl.ds(0, EPT_LAST)])
        pltpu.sync_copy(edst_hbm.at[pl.ds(15 * EPT, EPT_LAST)],
                        dst_v.at[pl.ds(0, EPT_LAST)])

    def _zero_body(i, carry):
        msg_v[pl.ds(i * 16, 16)] = zero16
        return carry
    lax.fori_loop(0, 9 * NPAD // (16 * 16), _zero_body, None)

    zchunk = 9 * NPAD // 16
    pltpu.sync_copy(msg_v.at[pl.ds(0, zchunk)],
                    shared_s.at[pl.ds(t * zchunk, zchunk)])

    plsc.subcore_barrier()

    def _ones_body(i, carry):
        msg_v[pl.ds(i * 16, 16)] = zero16 + 1.0
        return carry
    lax.fori_loop(0, EPT // 16, _ones_body, None)

    @pl.when(t < 15)
    def _():
        pltpu.sync_copy(msg_v.at[pl.ds(0, EPT)], srow(R_DEG, 0, NPAD).at[dst_v],
                        add=True)

    @pl.when(t == 15)
    def _():
        pltpu.sync_copy(msg_v.at[pl.ds(0, EPT_LAST)],
                        srow(R_DEG, 0, NPAD).at[dst_v.at[pl.ds(0, EPT_LAST)]],
                        add=True)

    plsc.subcore_barrier()

    pltpu.sync_copy(srow(R_DEG, base_n, RNG), dinv_v)

    lane = lax.iota(jnp.int32, 16)

    def _dinv_body(i, carry):
        d = dinv_v[pl.ds(i * 16, 16)] + 1.0
        bits = lax.bitcast_convert_type(d, jnp.int32)
        y = lax.bitcast_convert_type(
            jnp.int32(0x5F3759DF) - lax.shift_right_logical(bits, 1), F32)
        for _ in range(3):
            y = y * (1.5 - 0.5 * d * y * y)
        dinv_v[pl.ds(i * 16, 16)] = y
        nloc3 = (i * 16 + lane) * 3
        for c in range(3):
            pc = plsc.load_gather(posr_v, [nloc3 + c])
            gp_v[pl.ds(c * RNG + i * 16, 16)] = y * pc
        return carry
    lax.fori_loop(0, RNG // 16, _dinv_body, None)

    for c in range(3):
        pltpu.sync_copy(gp_v.at[pl.ds(c * RNG, RNG)],
                        srow(R_GP + c, base_n, RNG))

    plsc.subcore_barrier()

    def _msg_pass(rows_from, rows_to, nfeat):
        def gsrc(c):
            return srow(rows_from + c, 0, NPAD)

        def sdst(c, last):
            idx = dst_v.at[pl.ds(0, EPT_LAST)] if last else dst_v
            return srow(rows_to + c, 0, NPAD).at[idx]

        def mrow(c, last):
            return msg_v.at[pl.ds(c * NPAD, EPT_LAST if last else EPT)]

        @pl.when(t < 15)
        def _():
            for c in range(nfeat):
                pltpu.async_copy(gsrc(c).at[src_v], mrow(c, False), sem)
            for c in range(nfeat):
                pltpu.make_async_copy(gsrc(c).at[src_v], mrow(c, False),
                                      sem).wait()
            for c in range(nfeat):
                pltpu.async_copy(mrow(c, False), sdst(c, False), sem, add=True)
            for c in range(nfeat):
                pltpu.make_async_copy(mrow(c, False), sdst(c, False),
                                      sem).wait()

        @pl.when(t == 15)
        def _():
            srcl = src_v.at[pl.ds(0, EPT_LAST)]
            for c in range(nfeat):
                pltpu.async_copy(gsrc(c).at[srcl], mrow(c, True), sem)
            for c in range(nfeat):
                pltpu.make_async_copy(gsrc(c).at[srcl], mrow(c, True),
                                      sem).wait()
            for c in range(nfeat):
                pltpu.async_copy(mrow(c, True), sdst(c, True), sem, add=True)
            for c in range(nfeat):
                pltpu.make_async_copy(mrow(c, True), sdst(c, True), sem).wait()

    _msg_pass(R_GP, R_S1, 3)
    plsc.subcore_barrier()

    for c in range(3):
        pltpu.sync_copy(srow(R_S1 + c, base_n, RNG),
                        s1r_v.at[pl.ds(c * RNG, RNG)])

    def _x1_body(i, carry):
        dv = dinv_v[pl.ds(i * 16, 16)]
        ts = []
        for c in range(3):
            ts.append(dv * (s1r_v[pl.ds(c * RNG + i * 16, 16)]
                            + gp_v[pl.ds(c * RNG + i * 16, 16)]))
        for j in range(5):
            acc = sget(15 + j) + ts[0] * sget(j) \
                + ts[1] * sget(5 + j) + ts[2] * sget(10 + j)
            h_v[pl.ds(j * RNG + i * 16, 16)] = dv * jnp.maximum(acc, 0.01 * acc)
        return carry
    lax.fori_loop(0, RNG // 16, _x1_body, None)

    for c in range(5):
        pltpu.sync_copy(h_v.at[pl.ds(c * RNG, RNG)],
                        srow(R_H + c, base_n, RNG))

    plsc.subcore_barrier()

    _msg_pass(R_H, R_S2, 5)
    plsc.subcore_barrier()

    for c in range(5):
        pltpu.sync_copy(srow(R_S2 + c, base_n, RNG),
                        s2r_v.at[pl.ds(c * RNG, RNG)])

    def _fin_body(i, carry):
        dv = dinv_v[pl.ds(i * 16, 16)]
        ts = []
        for c in range(5):
            ts.append(dv * (s2r_v[pl.ds(c * RNG + i * 16, 16)]
                            + h_v[pl.ds(c * RNG + i * 16, 16)]))
        x2 = []
        for j in range(20):
            acc = sget(120 + j)
            for c in range(5):
                acc = acc + ts[c] * sget(20 + c * 20 + j)
            x2.append(jnp.maximum(acc, 0.01 * acc))
        nl10 = (i * 16 + lane) * 10
        for k in range(10):
            acc = sget(340 + k)
            for j in range(20):
                acc = acc + x2[j] * sget(140 + j * 10 + k)
            yv = jnp.maximum(acc, 0.01 * acc)
            plsc.store_scatter(y_v, [nl10 + k], yv)
        return carry
    lax.fori_loop(0, RNG // 16, _fin_body, None)

    pltpu.sync_copy(y_v, y_hbm.at[pl.ds(t * 3680, 3680)])


_RB = 4500


def _fc2_body(x_ref, w_ref, b_ref, o_ref):
    acc = b_ref[...]
    for i in range(13):
        acc = acc + jnp.dot(x_ref[i:i + 1, :], w_ref[_RB * i:_RB * (i + 1), :],
                            preferred_element_type=jnp.float32)
    o_ref[...] = jnp.maximum(acc, 0.01 * acc)


def _fc2(xf, W, b):
    out = pl.pallas_call(
        _fc2_body,
        out_shape=jax.ShapeDtypeStruct((1, 100), jnp.float32),
    )(xf, W, b.reshape(1, 100))
    return out[0]


def kernel(pos, edge_index, W1, b1, W2, b2, fc1_W, fc1_b, fc2_W, fc2_b):
    smalls = jnp.concatenate([
        W1.reshape(-1), b1, W2.reshape(-1), b2,
        fc1_W.reshape(-1), fc1_b, jnp.zeros((2,), jnp.float32)])
    y = _sc_graph(pos.reshape(-1), edge_index[0], edge_index[1], smalls)
    xf = y[:58500].reshape(13, 4500)
    return _fc2(xf, fc2_W, fc2_b)


# speedup vs baseline: 1.0548x; 1.0175x over previous
import functools

import jax
import jax.numpy as jnp
from jax import lax
from jax.experimental import pallas as pl
from jax.experimental.pallas import tpu as pltpu
from jax.experimental.pallas import tpu_sc as plsc

N = 5850
E = 93600
NPAD = 5888
RNG = 368
EPT = 5856
EPT_LAST = 5760
F32 = jnp.float32

R_DEG = 0
R_S1 = 1
R_S2 = 4
R_GP = 9
R_H = 12
NROWS = 17

_mesh = plsc.VectorSubcoreMesh(core_axis_name="c", subcore_axis_name="s",
                               num_cores=1)


@functools.partial(
    pl.kernel,
    out_type=jax.ShapeDtypeStruct((58880,), F32),
    mesh=_mesh,
    compiler_params=pltpu.CompilerParams(needs_layout_passes=False),
    scratch_types=[
        pltpu.VMEM((1104,), F32),
        pltpu.VMEM((368,), F32),
        pltpu.VMEM((1104,), F32),
        pltpu.VMEM((1104,), F32),
        pltpu.VMEM((1840,), F32),
        pltpu.VMEM((1840,), F32),
        pltpu.VMEM((5856,), jnp.int32),
        pltpu.VMEM((5856,), jnp.int32),
        pltpu.VMEM((29440,), F32),
        pltpu.VMEM((352,), F32),
        pltpu.VMEM((3680,), F32),
        pltpu.SemaphoreType.DMA,
        pltpu.VMEM_SHARED((NROWS * NPAD,), F32),
    ],
)
def _sc_graph(pos_hbm, edges_hbm, smalls_hbm, y_hbm,
              posr_v, dinv_v, gp_v, s1r_v, h_v, s2r_v, src_v, dst_v, msg_v,
              smalls_v, y_v, sem, shared_s):
    t = lax.axis_index("s")
    zero16 = jnp.zeros((16,), F32)
    base_n = t * RNG

    def sget(idx):
        return smalls_v[pl.ds((idx // 16) * 16, 16)][idx % 16]

    def srow(r, off, size):
        return shared_s.at[pl.ds(r * NPAD + off, size)]

    pltpu.sync_copy(smalls_hbm, smalls_v)

    @pl.when(t < 15)
    def _():
        pltpu.sync_copy(pos_hbm.at[pl.ds(t * 1104, 1104)], posr_v)
        pltpu.sync_copy(edges_hbm.at[pl.ds(t * EPT, EPT)], src_v)
        pltpu.sync_copy(edges_hbm.at[pl.ds(E + t * EPT, EPT)], dst_v)

    @pl.when(t == 15)
    def _():
        pltpu.sync_copy(pos_hbm.at[pl.ds(15 * 1104, 990)],
                        posr_v.at[pl.ds(0, 990)])
        pltpu.sync_copy(edges_hbm.at[pl.ds(15 * EPT, EPT_LAST)],
                        src_v.at[pl.ds(0, EPT_LAST)])
        pltpu.sync_copy(edges_hbm.at[pl.ds(E + 15 * EPT, EPT_LAST)],
                        dst_v.at[pl.ds(0, EPT_LAST)])

    def _zero_body(i, carry):
        msg_v[pl.ds(i * 16, 16)] = zero16
        return carry
    lax.fori_loop(0, 9 * NPAD // (16 * 16), _zero_body, None)

    zchunk = 9 * NPAD // 16
    pltpu.sync_copy(msg_v.at[pl.ds(0, zchunk)],
                    shared_s.at[pl.ds(t * zchunk, zchunk)])

    plsc.subcore_barrier()

    def _ones_body(i, carry):
        msg_v[pl.ds(i * 16, 16)] = zero16 + 1.0
        return carry
    lax.fori_loop(0, EPT // 16, _ones_body, None)

    @pl.when(t < 15)
    def _():
        pltpu.sync_copy(msg_v.at[pl.ds(0, EPT)], srow(R_DEG, 0, NPAD).at[dst_v],
                        add=True)

    @pl.when(t == 15)
    def _():
        pltpu.sync_copy(msg_v.at[pl.ds(0, EPT_LAST)],
                        srow(R_DEG, 0, NPAD).at[dst_v.at[pl.ds(0, EPT_LAST)]],
                        add=True)

    plsc.subcore_barrier()

    pltpu.sync_copy(srow(R_DEG, base_n, RNG), dinv_v)

    lane = lax.iota(jnp.int32, 16)

    def _dinv_body(i, carry):
        d = dinv_v[pl.ds(i * 16, 16)] + 1.0
        bits = lax.bitcast_convert_type(d, jnp.int32)
        y = lax.bitcast_convert_type(
            jnp.int32(0x5F3759DF) - lax.shift_right_logical(bits, 1), F32)
        for _ in range(3):
            y = y * (1.5 - 0.5 * d * y * y)
        dinv_v[pl.ds(i * 16, 16)] = y
        nloc3 = (i * 16 + lane) * 3
        for c in range(3):
            pc = plsc.load_gather(posr_v, [nloc3 + c])
            gp_v[pl.ds(c * RNG + i * 16, 16)] = y * pc
        return carry
    lax.fori_loop(0, RNG // 16, _dinv_body, None)

    for c in range(3):
        pltpu.sync_copy(gp_v.at[pl.ds(c * RNG, RNG)],
                        srow(R_GP + c, base_n, RNG))

    plsc.subcore_barrier()

    def _msg_pass(rows_from, rows_to, nfeat):
        def gsrc(c):
            return srow(rows_from + c, 0, NPAD)

        def sdst(c, last):
            idx = dst_v.at[pl.ds(0, EPT_LAST)] if last else dst_v
            return srow(rows_to + c, 0, NPAD).at[idx]

        def mrow(c, last):
            return msg_v.at[pl.ds(c * NPAD, EPT_LAST if last else EPT)]

        @pl.when(t < 15)
        def _():
            for c in range(nfeat):
                pltpu.async_copy(gsrc(c).at[src_v], mrow(c, False), sem)
            for c in range(nfeat):
                pltpu.make_async_copy(gsrc(c).at[src_v], mrow(c, False),
                                      sem).wait()
            for c in range(nfeat):
                pltpu.async_copy(mrow(c, False), sdst(c, False), sem, add=True)
            for c in range(nfeat):
                pltpu.make_async_copy(mrow(c, False), sdst(c, False),
                                      sem).wait()

        @pl.when(t == 15)
        def _():
            srcl = src_v.at[pl.ds(0, EPT_LAST)]
            for c in range(nfeat):
                pltpu.async_copy(gsrc(c).at[srcl], mrow(c, True), sem)
            for c in range(nfeat):
                pltpu.make_async_copy(gsrc(c).at[srcl], mrow(c, True),
                                      sem).wait()
            for c in range(nfeat):
                pltpu.async_copy(mrow(c, True), sdst(c, True), sem, add=True)
            for c in range(nfeat):
                pltpu.make_async_copy(mrow(c, True), sdst(c, True), sem).wait()

    _msg_pass(R_GP, R_S1, 3)
    plsc.subcore_barrier()

    for c in range(3):
        pltpu.sync_copy(srow(R_S1 + c, base_n, RNG),
                        s1r_v.at[pl.ds(c * RNG, RNG)])

    def _x1_body(i, carry):
        dv = dinv_v[pl.ds(i * 16, 16)]
        ts = []
        for c in range(3):
            ts.append(dv * (s1r_v[pl.ds(c * RNG + i * 16, 16)]
                            + gp_v[pl.ds(c * RNG + i * 16, 16)]))
        for j in range(5):
            acc = sget(15 + j) + ts[0] * sget(j) \
                + ts[1] * sget(5 + j) + ts[2] * sget(10 + j)
            h_v[pl.ds(j * RNG + i * 16, 16)] = dv * jnp.maximum(acc, 0.01 * acc)
        return carry
    lax.fori_loop(0, RNG // 16, _x1_body, None)

    for c in range(5):
        pltpu.sync_copy(h_v.at[pl.ds(c * RNG, RNG)],
                        srow(R_H + c, base_n, RNG))

    plsc.subcore_barrier()

    _msg_pass(R_H, R_S2, 5)
    plsc.subcore_barrier()

    for c in range(5):
        pltpu.sync_copy(srow(R_S2 + c, base_n, RNG),
                        s2r_v.at[pl.ds(c * RNG, RNG)])

    def _fin_body(i, carry):
        dv = dinv_v[pl.ds(i * 16, 16)]
        ts = []
        for c in range(5):
            ts.append(dv * (s2r_v[pl.ds(c * RNG + i * 16, 16)]
                            + h_v[pl.ds(c * RNG + i * 16, 16)]))
        x2 = []
        for j in range(20):
            acc = sget(120 + j)
            for c in range(5):
                acc = acc + ts[c] * sget(20 + c * 20 + j)
            x2.append(jnp.maximum(acc, 0.01 * acc))
        nl10 = (i * 16 + lane) * 10
        for k in range(10):
            acc = sget(340 + k)
            for j in range(20):
                acc = acc + x2[j] * sget(140 + j * 10 + k)
            yv = jnp.maximum(acc, 0.01 * acc)
            plsc.store_scatter(y_v, [nl10 + k], yv)
        return carry
    lax.fori_loop(0, RNG // 16, _fin_body, None)

    pltpu.sync_copy(y_v, y_hbm.at[pl.ds(t * 3680, 3680)])


_RB = 4500


def _fc2_body(x_ref, w_ref, b_ref, o_ref):
    acc = b_ref[...]
    for i in range(13):
        acc = acc + jnp.dot(x_ref[i:i + 1, :], w_ref[_RB * i:_RB * (i + 1), :],
                            preferred_element_type=jnp.float32)
    o_ref[...] = jnp.maximum(acc, 0.01 * acc)


def _fc2(xf, W, b):
    out = pl.pallas_call(
        _fc2_body,
        out_shape=jax.ShapeDtypeStruct((1, 100), jnp.float32),
    )(xf, W, b.reshape(1, 100))
    return out[0]


def kernel(pos, edge_index, W1, b1, W2, b2, fc1_W, fc1_b, fc2_W, fc2_b):
    smalls = jnp.concatenate([
        W1.reshape(-1), b1, W2.reshape(-1), b2,
        fc1_W.reshape(-1), fc1_b, jnp.zeros((2,), jnp.float32)])
    y = _sc_graph(pos.reshape(-1), edge_index.reshape(-1), smalls)
    xf = y[:58500].reshape(13, 4500)
    return _fc2(xf, fc2_W, fc2_b)


# speedup vs baseline: 1.0612x; 1.0061x over previous
import functools

import jax
import jax.numpy as jnp
from jax import lax
from jax.experimental import pallas as pl
from jax.experimental.pallas import tpu as pltpu
from jax.experimental.pallas import tpu_sc as plsc

N = 5850
E = 93600
NPAD = 5888
RNG = 368
EPT = 5856
EPT_LAST = 5760
F32 = jnp.float32

R_DEG = 0
R_S1 = 1
R_S2 = 4
R_GP = 9
R_H = 12
NROWS = 17

_mesh = plsc.VectorSubcoreMesh(core_axis_name="c", subcore_axis_name="s",
                               num_cores=1)


@functools.partial(
    pl.kernel,
    out_type=jax.ShapeDtypeStruct((58880,), F32),
    mesh=_mesh,
    compiler_params=pltpu.CompilerParams(needs_layout_passes=False),
    scratch_types=[
        pltpu.VMEM((1104,), F32),
        pltpu.VMEM((368,), F32),
        pltpu.VMEM((1104,), F32),
        pltpu.VMEM((1104,), F32),
        pltpu.VMEM((1840,), F32),
        pltpu.VMEM((1840,), F32),
        pltpu.VMEM((5856,), jnp.int32),
        pltpu.VMEM((5856,), jnp.int32),
        pltpu.VMEM((29440,), F32),
        pltpu.VMEM((352,), F32),
        pltpu.VMEM((3680,), F32),
        pltpu.SemaphoreType.DMA,
        pltpu.VMEM_SHARED((NROWS * NPAD,), F32),
    ],
)
def _sc_graph(ps_hbm, edges_hbm, y_hbm,
              posr_v, dinv_v, gp_v, s1r_v, h_v, s2r_v, src_v, dst_v, msg_v,
              smalls_v, y_v, sem, shared_s):
    t = lax.axis_index("s")
    zero16 = jnp.zeros((16,), F32)
    base_n = t * RNG

    def sget(idx):
        return smalls_v[pl.ds((idx // 16) * 16, 16)][idx % 16]

    def srow(r, off, size):
        return shared_s.at[pl.ds(r * NPAD + off, size)]

    pltpu.sync_copy(ps_hbm.at[pl.ds(17552, 352)], smalls_v)

    @pl.when(t < 15)
    def _():
        pltpu.sync_copy(ps_hbm.at[pl.ds(t * 1104, 1104)], posr_v)
        pltpu.sync_copy(edges_hbm.at[pl.ds(t * EPT, EPT)], src_v)
        pltpu.sync_copy(edges_hbm.at[pl.ds(E + t * EPT, EPT)], dst_v)

    @pl.when(t == 15)
    def _():
        pltpu.sync_copy(ps_hbm.at[pl.ds(15 * 1104, 990)],
                        posr_v.at[pl.ds(0, 990)])
        pltpu.sync_copy(edges_hbm.at[pl.ds(15 * EPT, EPT_LAST)],
                        src_v.at[pl.ds(0, EPT_LAST)])
        pltpu.sync_copy(edges_hbm.at[pl.ds(E + 15 * EPT, EPT_LAST)],
                        dst_v.at[pl.ds(0, EPT_LAST)])

    def _zero_body(i, carry):
        msg_v[pl.ds(i * 16, 16)] = zero16
        return carry
    lax.fori_loop(0, 9 * NPAD // (16 * 16), _zero_body, None)

    zchunk = 9 * NPAD // 16
    pltpu.sync_copy(msg_v.at[pl.ds(0, zchunk)],
                    shared_s.at[pl.ds(t * zchunk, zchunk)])

    plsc.subcore_barrier()

    def _ones_body(i, carry):
        msg_v[pl.ds(i * 16, 16)] = zero16 + 1.0
        return carry
    lax.fori_loop(0, EPT // 16, _ones_body, None)

    @pl.when(t < 15)
    def _():
        pltpu.sync_copy(msg_v.at[pl.ds(0, EPT)], srow(R_DEG, 0, NPAD).at[dst_v],
                        add=True)

    @pl.when(t == 15)
    def _():
        pltpu.sync_copy(msg_v.at[pl.ds(0, EPT_LAST)],
                        srow(R_DEG, 0, NPAD).at[dst_v.at[pl.ds(0, EPT_LAST)]],
                        add=True)

    plsc.subcore_barrier()

    pltpu.sync_copy(srow(R_DEG, base_n, RNG), dinv_v)

    lane = lax.iota(jnp.int32, 16)

    def _dinv_body(i, carry):
        d = dinv_v[pl.ds(i * 16, 16)] + 1.0
        bits = lax.bitcast_convert_type(d, jnp.int32)
        y = lax.bitcast_convert_type(
            jnp.int32(0x5F3759DF) - lax.shift_right_logical(bits, 1), F32)
        for _ in range(3):
            y = y * (1.5 - 0.5 * d * y * y)
        dinv_v[pl.ds(i * 16, 16)] = y
        nloc3 = (i * 16 + lane) * 3
        for c in range(3):
            pc = plsc.load_gather(posr_v, [nloc3 + c])
            gp_v[pl.ds(c * RNG + i * 16, 16)] = y * pc
        return carry
    lax.fori_loop(0, RNG // 16, _dinv_body, None)

    for c in range(3):
        pltpu.sync_copy(gp_v.at[pl.ds(c * RNG, RNG)],
                        srow(R_GP + c, base_n, RNG))

    plsc.subcore_barrier()

    def _msg_pass(rows_from, rows_to, nfeat):
        def gsrc(c):
            return srow(rows_from + c, 0, NPAD)

        def sdst(c, last):
            idx = dst_v.at[pl.ds(0, EPT_LAST)] if last else dst_v
            return srow(rows_to + c, 0, NPAD).at[idx]

        def mrow(c, last):
            return msg_v.at[pl.ds(c * NPAD, EPT_LAST if last else EPT)]

        @pl.when(t < 15)
        def _():
            for c in range(nfeat):
                pltpu.async_copy(gsrc(c).at[src_v], mrow(c, False), sem)
            for c in range(nfeat):
                pltpu.make_async_copy(gsrc(c).at[src_v], mrow(c, False),
                                      sem).wait()
            for c in range(nfeat):
                pltpu.async_copy(mrow(c, False), sdst(c, False), sem, add=True)
            for c in range(nfeat):
                pltpu.make_async_copy(mrow(c, False), sdst(c, False),
                                      sem).wait()

        @pl.when(t == 15)
        def _():
            srcl = src_v.at[pl.ds(0, EPT_LAST)]
            for c in range(nfeat):
                pltpu.async_copy(gsrc(c).at[srcl], mrow(c, True), sem)
            for c in range(nfeat):
                pltpu.make_async_copy(gsrc(c).at[srcl], mrow(c, True),
                                      sem).wait()
            for c in range(nfeat):
                pltpu.async_copy(mrow(c, True), sdst(c, True), sem, add=True)
            for c in range(nfeat):
                pltpu.make_async_copy(mrow(c, True), sdst(c, True), sem).wait()

    _msg_pass(R_GP, R_S1, 3)
    plsc.subcore_barrier()

    for c in range(3):
        pltpu.sync_copy(srow(R_S1 + c, base_n, RNG),
                        s1r_v.at[pl.ds(c * RNG, RNG)])

    def _x1_body(i, carry):
        dv = dinv_v[pl.ds(i * 16, 16)]
        ts = []
        for c in range(3):
            ts.append(dv * (s1r_v[pl.ds(c * RNG + i * 16, 16)]
                            + gp_v[pl.ds(c * RNG + i * 16, 16)]))
        for j in range(5):
            acc = sget(15 + j) + ts[0] * sget(j) \
                + ts[1] * sget(5 + j) + ts[2] * sget(10 + j)
            h_v[pl.ds(j * RNG + i * 16, 16)] = dv * jnp.maximum(acc, 0.01 * acc)
        return carry
    lax.fori_loop(0, RNG // 16, _x1_body, None)

    for c in range(5):
        pltpu.sync_copy(h_v.at[pl.ds(c * RNG, RNG)],
                        srow(R_H + c, base_n, RNG))

    plsc.subcore_barrier()

    _msg_pass(R_H, R_S2, 5)
    plsc.subcore_barrier()

    for c in range(5):
        pltpu.sync_copy(srow(R_S2 + c, base_n, RNG),
                        s2r_v.at[pl.ds(c * RNG, RNG)])

    def _fin_body(i, carry):
        dv = dinv_v[pl.ds(i * 16, 16)]
        ts = []
        for c in range(5):
            ts.append(dv * (s2r_v[pl.ds(c * RNG + i * 16, 16)]
                            + h_v[pl.ds(c * RNG + i * 16, 16)]))
        x2 = []
        for j in range(20):
            acc = sget(120 + j)
            for c in range(5):
                acc = acc + ts[c] * sget(20 + c * 20 + j)
            x2.append(jnp.maximum(acc, 0.01 * acc))
        nl10 = (i * 16 + lane) * 10
        for k in range(10):
            acc = sget(340 + k)
            for j in range(20):
                acc = acc + x2[j] * sget(140 + j * 10 + k)
            yv = jnp.maximum(acc, 0.01 * acc)
            plsc.store_scatter(y_v, [nl10 + k], yv)
        return carry
    lax.fori_loop(0, RNG // 16, _fin_body, None)

    pltpu.sync_copy(y_v, y_hbm.at[pl.ds(t * 3680, 3680)])


_RB = 4500


def _fc2_body(x_ref, w_ref, b_ref, o_ref):
    acc = b_ref[...]
    for i in range(13):
        acc = acc + jnp.dot(x_ref[i:i + 1, :], w_ref[_RB * i:_RB * (i + 1), :],
                            preferred_element_type=jnp.float32)
    o_ref[...] = jnp.maximum(acc, 0.01 * acc)


def _fc2(xf, W, b):
    out = pl.pallas_call(
        _fc2_body,
        out_shape=jax.ShapeDtypeStruct((1, 100), jnp.float32),
    )(xf, W, b.reshape(1, 100))
    return out[0]


def kernel(pos, edge_index, W1, b1, W2, b2, fc1_W, fc1_b, fc2_W, fc2_b):
    ps = jnp.concatenate([
        pos.reshape(-1), jnp.zeros((2,), jnp.float32),
        W1.reshape(-1), b1, W2.reshape(-1), b2,
        fc1_W.reshape(-1), fc1_b, jnp.zeros((2,), jnp.float32)])
    y = _sc_graph(ps, edge_index.reshape(-1))
    xf = y[:58500].reshape(13, 4500)
    return _fc2(xf, fc2_W, fc2_b)
